# baseline (speedup 1.0000x reference)
import functools

import jax
import jax.numpy as jnp
from jax import lax
from jax.experimental import pallas as pl
from jax.experimental.pallas import tpu as pltpu
from jax.experimental.pallas import tpu_sc as plsc


def _matmul_kernel(x_ref, w_ref, gate_ref):
    gate_ref[...] = jnp.dot(
        x_ref[...], w_ref[...], preferred_element_type=jnp.float32
    )


def _make_router(N, E, b_per_w):
    mesh = plsc.VectorSubcoreMesh(core_axis_name="c", subcore_axis_name="s")

    @functools.partial(
        pl.kernel,
        mesh=mesh,
        out_type=[
            jax.ShapeDtypeStruct((N, E), jnp.float32),
            jax.ShapeDtypeStruct((N, E), jnp.int32),
        ],
        scratch_types=[
            pltpu.VMEM((b_per_w, E), jnp.float32),
            pltpu.VMEM((b_per_w, E), jnp.float32),
            pltpu.VMEM((b_per_w, E), jnp.int32),
        ],
        compiler_params=pltpu.CompilerParams(needs_layout_passes=False, use_tc_tiling_on_sc=False),
    )
    def route(logits_hbm, probs_hbm, idx_hbm, lg_v, pr_v, ix_v):
        nc = 2
        wid = lax.axis_index("s") * nc + lax.axis_index("c")
        base = wid * b_per_w
        pltpu.sync_copy(logits_hbm.at[pl.ds(base, b_per_w)], lg_v)
        iota = lax.iota(jnp.int32, E)

        def body(t, carry):
            v = lg_v[t, :]
            top1 = jnp.max(v)
            i1 = jnp.min(jnp.where(v == top1, iota, E))
            masked = jnp.where(iota == i1, -jnp.inf, v)
            top2 = jnp.max(masked)
            i2 = jnp.min(jnp.where(masked == top2, iota, E))
            mask2 = (iota == i1) | (iota == i2)
            e = jnp.exp(v - top1)
            s = jnp.sum(jnp.where(mask2, e, 0.0))
            pr_v[t, :] = jnp.where(mask2, e / s, 0.0)
            ix_v[t, :] = jnp.where(iota == 0, i1, jnp.where(iota == 1, i2, 0))
            return carry

        lax.fori_loop(0, b_per_w, body, 0)
        pltpu.sync_copy(pr_v, probs_hbm.at[pl.ds(base, b_per_w)])
        pltpu.sync_copy(ix_v, idx_hbm.at[pl.ds(base, b_per_w)])

    return route


def kernel(x, W):
    B, S, H = x.shape
    E = W.shape[0]
    K = 2
    N = B * S
    T = 2048
    xf = x.reshape(N, H)
    wt = W.T

    gate = pl.pallas_call(
        _matmul_kernel,
        grid=(N // T,),
        in_specs=[
            pl.BlockSpec((T, H), lambda i: (i, 0)),
            pl.BlockSpec((H, E), lambda i: (0, 0)),
        ],
        out_specs=pl.BlockSpec((T, E), lambda i: (i, 0)),
        out_shape=jax.ShapeDtypeStruct((N, E), jnp.float32),
        compiler_params=pltpu.CompilerParams(
            dimension_semantics=("arbitrary",),
        ),
    )(xf, wt)

    route = _make_router(N, E, N // 32)
    probs, idx16 = route(gate)
    idx = idx16[:, :K]
    return probs.reshape(B, S, E), idx.reshape(B, S, K), gate


# speedup vs baseline: 1.4167x; 1.4167x over previous
import jax
import jax.numpy as jnp
from jax.experimental import pallas as pl
from jax.experimental.pallas import tpu as pltpu


def _router_kernel(x_ref, w_ref, gate_ref, probs_ref, idx_ref):
    T, E = gate_ref.shape
    logits = jnp.dot(x_ref[...], w_ref[...], preferred_element_type=jnp.float32)
    gate_ref[...] = logits
    lt = logits.T
    iota = jax.lax.broadcasted_iota(jnp.int32, (E, T), 0)
    top1 = jnp.max(lt, axis=0, keepdims=True)
    i1 = jnp.min(jnp.where(lt == top1, iota, E), axis=0, keepdims=True)
    masked = jnp.where(iota == i1, -jnp.inf, lt)
    top2 = jnp.max(masked, axis=0, keepdims=True)
    i2 = jnp.min(jnp.where(masked == top2, iota, E), axis=0, keepdims=True)
    t = jnp.exp(top2 - top1)
    p1 = 1.0 / (1.0 + t)
    p2 = t / (1.0 + t)
    probs_t = jnp.where(iota == i1, p1, jnp.where(iota == i2, p2, 0.0))
    probs_ref[...] = probs_t.T
    idx_t = jnp.where(iota == 0, i1, jnp.where(iota == 1, i2, 0))
    idx_ref[...] = idx_t.T[:, : idx_ref.shape[1]]


def kernel(x, W):
    B, S, H = x.shape
    E = W.shape[0]
    K = 2
    N = B * S
    T = 1024
    xf = x.reshape(N, H)
    wt = W.T

    gate, probs, idx = pl.pallas_call(
        _router_kernel,
        grid=(N // T,),
        in_specs=[
            pl.BlockSpec((T, H), lambda i: (i, 0)),
            pl.BlockSpec((H, E), lambda i: (0, 0)),
        ],
        out_specs=[
            pl.BlockSpec((T, E), lambda i: (i, 0)),
            pl.BlockSpec((T, E), lambda i: (i, 0)),
            pl.BlockSpec((T, K), lambda i: (i, 0)),
        ],
        out_shape=[
            jax.ShapeDtypeStruct((N, E), jnp.float32),
            jax.ShapeDtypeStruct((N, E), jnp.float32),
            jax.ShapeDtypeStruct((N, K), jnp.int32),
        ],
        compiler_params=pltpu.CompilerParams(
            dimension_semantics=("arbitrary",),
        ),
    )(xf, wt)
    return probs.reshape(B, S, E), idx.reshape(B, S, K), gate


# speedup vs baseline: 1.4214x; 1.0034x over previous
import jax
import jax.numpy as jnp
from jax.experimental import pallas as pl
from jax.experimental.pallas import tpu as pltpu


def _router_kernel(x_ref, w_ref, gate_ref, probs_ref, idx_ref):
    T, E = gate_ref.shape
    logits = jnp.dot(x_ref[...], w_ref[...], preferred_element_type=jnp.float32)
    gate_ref[...] = logits
    lt = logits.T
    iota = jax.lax.broadcasted_iota(jnp.int32, (E, T), 0)
    top1 = jnp.max(lt, axis=0, keepdims=True)
    i1 = jnp.min(jnp.where(lt == top1, iota, E), axis=0, keepdims=True)
    masked = jnp.where(iota == i1, -jnp.inf, lt)
    top2 = jnp.max(masked, axis=0, keepdims=True)
    i2 = jnp.min(jnp.where(masked == top2, iota, E), axis=0, keepdims=True)
    t = jnp.exp(top2 - top1)
    p1 = 1.0 / (1.0 + t)
    p2 = t / (1.0 + t)
    probs_t = jnp.where(iota == i1, p1, jnp.where(iota == i2, p2, 0.0))
    probs_ref[...] = probs_t.T
    idx_t = jnp.where(iota == 0, i1, jnp.where(iota == 1, i2, 0))
    idx_ref[...] = idx_t.T[:, : idx_ref.shape[1]]


def kernel(x, W):
    B, S, H = x.shape
    E = W.shape[0]
    K = 2
    N = B * S
    T = 2048
    xf = x.reshape(N, H)
    wt = W.T

    gate, probs, idx = pl.pallas_call(
        _router_kernel,
        grid=(N // T,),
        in_specs=[
            pl.BlockSpec((T, H), lambda i: (i, 0)),
            pl.BlockSpec((H, E), lambda i: (0, 0)),
        ],
        out_specs=[
            pl.BlockSpec((T, E), lambda i: (i, 0)),
            pl.BlockSpec((T, E), lambda i: (i, 0)),
            pl.BlockSpec((T, K), lambda i: (i, 0)),
        ],
        out_shape=[
            jax.ShapeDtypeStruct((N, E), jnp.float32),
            jax.ShapeDtypeStruct((N, E), jnp.float32),
            jax.ShapeDtypeStruct((N, K), jnp.int32),
        ],
        compiler_params=pltpu.CompilerParams(
            dimension_semantics=("arbitrary",),
        ),
    )(xf, wt)
    return probs.reshape(B, S, E), idx.reshape(B, S, K), gate
